# Initial kernel scaffold; baseline (speedup 1.0000x reference)
import jax
import jax.numpy as jnp
from jax.experimental import pallas as pl


def kernel(x, edge_index, batch, W_enc, b_enc, ggnn_w, gru_w_ih, gru_w_hh, gru_b_ih, gru_b_hh, c_w1, c_b1, c_w2, c_b2, c_w3, c_b3):
    raise NotImplementedError("write your pallas kernel here")


# speedup vs baseline: 2.3958x; 2.3958x over previous
import functools

import jax
import jax.numpy as jnp
from jax import lax
from jax.experimental import pallas as pl
from jax.experimental.pallas import tpu as pltpu
from jax.experimental.pallas import tpu_sc as plsc

N = 10000
E = 320000
EMB = 256
CD = 101
H = 128
G = 64
L = 3

NC = 2
NS = 16
NW = NC * NS
CH = 128
NCHUNK = 80
EPW = CH * NCHUNK
EPAD = EPW * NW
NP = 10112
ZR = NP // NS

def _edge_agg_body(m_hbm, src_hbm, dst_hbm, zeros_hbm, out_hbm,
                   src_v, dst_v, rows_v, acc_sh, sem):
    c = lax.axis_index("c")
    s = lax.axis_index("s")
    wid = c * NS + s
    pltpu.sync_copy(zeros_hbm, acc_sh.at[pl.ds(s * ZR, ZR)])
    plsc.subcore_barrier()

    def body(j, _):
        base = pl.multiple_of(wid * EPW + j * CH, CH)
        pltpu.sync_copy(src_hbm.at[pl.ds(base, CH)], src_v)
        pltpu.sync_copy(dst_hbm.at[pl.ds(base, CH)], dst_v)
        pltpu.async_copy(m_hbm.at[src_v], rows_v, sem).wait()
        pltpu.sync_copy(rows_v, acc_sh.at[dst_v], add=True)
        return 0

    lax.fori_loop(0, NCHUNK, body, 0)
    plsc.subcore_barrier()
    pltpu.sync_copy(acc_sh.at[pl.ds(s * ZR, ZR)],
                    out_hbm.at[c, pl.ds(s * ZR, ZR)])


@functools.cache
def _make_edge_agg():
    mesh = plsc.VectorSubcoreMesh(core_axis_name="c", subcore_axis_name="s")
    return pl.kernel(
        _edge_agg_body,
        out_type=jax.ShapeDtypeStruct((NC, NP, H), jnp.float32),
        mesh=mesh,
        scratch_types=[
            pltpu.VMEM((CH,), jnp.int32),
            pltpu.VMEM((CH,), jnp.int32),
            pltpu.VMEM((CH, H), jnp.float32),
            pltpu.VMEM_SHARED((NP, H), jnp.float32),
            pltpu.SemaphoreType.DMA,
        ],
    )


def _edge_agg(m, src, dst, zeros_blk):
    return _make_edge_agg()(m, src, dst, zeros_blk)


BN = 1000
NBLK = N // BN


def _pre_body(x_ref, wenc_ref, benc_ref, w0_ref, h0_ref, m0_ref):
    h0 = jnp.maximum(
        jnp.dot(x_ref[...], wenc_ref[...], preferred_element_type=jnp.float32)
        + benc_ref[...], 0.0)
    h0_ref[...] = h0
    m0_ref[...] = jnp.dot(h0, w0_ref[...], preferred_element_type=jnp.float32)


def _pre(x, wencp, bencp, w0):
    return pl.pallas_call(
        _pre_body,
        grid=(NBLK,),
        in_specs=[
            pl.BlockSpec((BN, EMB), lambda i: (i, 0)),
            pl.BlockSpec((EMB, H), lambda i: (0, 0)),
            pl.BlockSpec((1, H), lambda i: (0, 0)),
            pl.BlockSpec((H, H), lambda i: (0, 0)),
        ],
        out_specs=[
            pl.BlockSpec((BN, H), lambda i: (i, 0)),
            pl.BlockSpec((BN, H), lambda i: (i, 0)),
        ],
        out_shape=[
            jax.ShapeDtypeStruct((N, H), jnp.float32),
            jax.ShapeDtypeStruct((N, H), jnp.float32),
        ],
    )(x, wencp, bencp, w0)


def _gru_body(compute_m, aggp_ref, h_ref, wih_ref, whh_ref, bih_ref, bhh_ref,
              wnext_ref, h_out_ref, m_out_ref):
    agg = aggp_ref[0] + aggp_ref[1]
    h = h_ref[...]
    gi = jnp.dot(agg, wih_ref[...], preferred_element_type=jnp.float32) + bih_ref[...]
    gh = jnp.dot(h, whh_ref[...], preferred_element_type=jnp.float32) + bhh_ref[...]
    r = jax.nn.sigmoid(gi[:, :H] + gh[:, :H])
    z = jax.nn.sigmoid(gi[:, H:2 * H] + gh[:, H:2 * H])
    n = jnp.tanh(gi[:, 2 * H:] + r * gh[:, 2 * H:])
    h_new = (1.0 - z) * n + z * h
    h_out_ref[...] = h_new
    if compute_m:
        m_out_ref[...] = jnp.dot(h_new, wnext_ref[...],
                                 preferred_element_type=jnp.float32)
    else:
        m_out_ref[...] = h_new


def _gru(aggp, h, wih_t, whh_t, bih, bhh, wnext, compute_m):
    return pl.pallas_call(
        functools.partial(_gru_body, compute_m),
        grid=(NBLK,),
        in_specs=[
            pl.BlockSpec((NC, BN, H), lambda i: (0, i, 0)),
            pl.BlockSpec((BN, H), lambda i: (i, 0)),
            pl.BlockSpec((H, 3 * H), lambda i: (0, 0)),
            pl.BlockSpec((H, 3 * H), lambda i: (0, 0)),
            pl.BlockSpec((1, 3 * H), lambda i: (0, 0)),
            pl.BlockSpec((1, 3 * H), lambda i: (0, 0)),
            pl.BlockSpec((H, H), lambda i: (0, 0)),
        ],
        out_specs=[
            pl.BlockSpec((BN, H), lambda i: (i, 0)),
            pl.BlockSpec((BN, H), lambda i: (i, 0)),
        ],
        out_shape=[
            jax.ShapeDtypeStruct((N, H), jnp.float32),
            jax.ShapeDtypeStruct((N, H), jnp.float32),
        ],
    )(aggp, h, wih_t, whh_t, bih, bhh, wnext)


def _pool_body(h_ref, c_ref, b_ref, w1h_ref, w1c_ref, b1_ref, w2_ref, b2_ref,
               w3_ref, b3_ref, out_ref, acc_h, acc_c, acc_n):
    i = pl.program_id(0)

    @pl.when(i == 0)
    def _init():
        acc_h[...] = jnp.zeros_like(acc_h)
        acc_c[...] = jnp.zeros_like(acc_c)
        acc_n[...] = jnp.zeros_like(acc_n)

    bb = b_ref[0, 0, :]
    seg = lax.broadcasted_iota(jnp.int32, (BN, G), 1)
    onehot = (bb[:, None] == seg).astype(jnp.float32)
    acc_h[...] += lax.dot_general(onehot, h_ref[...], (((0,), (0,)), ((), ())),
                                  preferred_element_type=jnp.float32)
    acc_c[...] += lax.dot_general(onehot, c_ref[...], (((0,), (0,)), ((), ())),
                                  preferred_element_type=jnp.float32)
    acc_n[...] += jnp.sum(onehot, axis=0)[None, :]

    @pl.when(i == NBLK - 1)
    def _finish():
        inv = 1.0 / jnp.maximum(acc_n[0, :], 1.0)
        gh = acc_h[...] * inv[:, None]
        gc = acc_c[...] * inv[:, None]
        h1 = jnp.maximum(
            jnp.dot(gh, w1h_ref[...], preferred_element_type=jnp.float32)
            + jnp.dot(gc, w1c_ref[...], preferred_element_type=jnp.float32)
            + b1_ref[...], 0.0)
        h2 = jnp.maximum(
            jnp.dot(h1, w2_ref[...], preferred_element_type=jnp.float32)
            + b2_ref[...], 0.0)
        logits = jnp.dot(h2, w3_ref[...], preferred_element_type=jnp.float32) \
            + b3_ref[...]
        out_ref[...] = jax.nn.sigmoid(logits)


def _pool_mlp(h, c, batch3, w1h, w1c, b1, w2, b2, w3p, b3p):
    return pl.pallas_call(
        _pool_body,
        grid=(NBLK,),
        in_specs=[
            pl.BlockSpec((BN, H), lambda i: (i, 0)),
            pl.BlockSpec((BN, H), lambda i: (i, 0)),
            pl.BlockSpec((1, 1, BN), lambda i: (i, 0, 0)),
            pl.BlockSpec((H, 256), lambda i: (0, 0)),
            pl.BlockSpec((H, 256), lambda i: (0, 0)),
            pl.BlockSpec((1, 256), lambda i: (0, 0)),
            pl.BlockSpec((256, H), lambda i: (0, 0)),
            pl.BlockSpec((1, H), lambda i: (0, 0)),
            pl.BlockSpec((H, H), lambda i: (0, 0)),
            pl.BlockSpec((1, H), lambda i: (0, 0)),
        ],
        out_specs=pl.BlockSpec((G, H), lambda i: (0, 0)),
        out_shape=jax.ShapeDtypeStruct((G, H), jnp.float32),
        scratch_shapes=[
            pltpu.VMEM((G, H), jnp.float32),
            pltpu.VMEM((G, H), jnp.float32),
            pltpu.VMEM((1, G), jnp.float32),
        ],
    )(h, c, batch3, w1h, w1c, b1, w2, b2, w3p, b3p)


def kernel(x, edge_index, batch, W_enc, b_enc, ggnn_w, gru_w_ih, gru_w_hh,
           gru_b_ih, gru_b_hh, c_w1, c_b1, c_w2, c_b2, c_w3, c_b3):
    f32 = jnp.float32
    wencp = jnp.pad(W_enc, ((0, 0), (0, H - CD)))
    bencp = jnp.pad(b_enc, (0, H - CD)).reshape(1, H)
    wih_t = gru_w_ih.T
    whh_t = gru_w_hh.T
    bih = gru_b_ih.reshape(1, 3 * H)
    bhh = gru_b_hh.reshape(1, 3 * H)
    w1h = c_w1[:, :H].T
    w1c = jnp.pad(c_w1[:, H:], ((0, 0), (0, H - CD))).T
    b1 = c_b1.reshape(1, 256)
    w2 = c_w2.T
    b2 = c_b2.reshape(1, H)
    w3p = jnp.pad(c_w3, ((0, H - 1), (0, 0))).T
    b3p = jnp.pad(c_b3, (0, H - 1)).reshape(1, H)

    src = jnp.concatenate([edge_index[0],
                           jnp.zeros((EPAD - E,), jnp.int32)])
    dst = jnp.concatenate([edge_index[1],
                           jnp.full((EPAD - E,), N, jnp.int32)])
    zeros_blk = jnp.zeros((ZR, H), f32)
    batch3 = batch.reshape(NBLK, 1, BN)

    h, m = _pre(x, wencp, bencp, ggnn_w[0])
    h0 = h
    for i in range(L):
        aggp = _edge_agg(m, src, dst, zeros_blk)
        h, m = _gru(aggp, h, wih_t, whh_t, bih, bhh,
                    ggnn_w[(i + 1) % L], compute_m=(i < L - 1))
    out = _pool_mlp(h, h0, batch3, w1h, w1c, b1, w2, b2, w3p, b3p)
    return out[:, :1]
